# baseline (speedup 1.0000x reference)
import functools

import jax
import jax.numpy as jnp
import numpy as np
from jax import lax
from jax.experimental import pallas as pl
from jax.experimental.pallas import tpu as pltpu
from jax.experimental.pallas import tpu_sc as plsc

B = 16384
L = 20
D = 32
NUMF = 26
HID = 256
CONCAT = 896
TB = 128
NT = B // TB
TW = TB * L


def _sc_gather(word_tab, widx, city_tab, cidx, nb):
    info = plsc.get_sparse_core_info()
    NC, NS = info.num_cores, info.num_subcores
    NW = NC * NS
    WPW = (nb * L) // NW
    CH = min(WPW, 2048)
    NCH = WPW // CH
    CPW = (2 * nb) // NW

    mesh = plsc.VectorSubcoreMesh(core_axis_name="c", subcore_axis_name="s")

    @functools.partial(
        pl.kernel,
        mesh=mesh,
        compiler_params=pltpu.CompilerParams(use_tc_tiling_on_sc=False),
        out_type=[
            jax.ShapeDtypeStruct((nb * L, D), jnp.float32),
            jax.ShapeDtypeStruct((2 * nb, D), jnp.float32),
        ],
        scratch_types=[
            pltpu.VMEM((CH,), jnp.int32),
            pltpu.VMEM((CH, D), jnp.float32),
            pltpu.VMEM((CPW,), jnp.int32),
            pltpu.VMEM((CPW, D), jnp.float32),
            pltpu.SemaphoreType.DMA,
        ],
    )
    def gather_kernel(wtab, wi, ctab, ci, wout, cout,
                      idx_v, rows_v, cidx_v, crows_v, sem):
        wid = lax.axis_index("s") * NC + lax.axis_index("c")
        base = wid * WPW
        for c in range(NCH):
            off = base + c * CH
            pltpu.sync_copy(wi.at[pl.ds(off, CH)], idx_v)
            pltpu.async_copy(wtab.at[idx_v], rows_v, sem).wait()
            pltpu.sync_copy(rows_v, wout.at[pl.ds(off, CH)])
        cbase = wid * CPW
        pltpu.sync_copy(ci.at[pl.ds(cbase, CPW)], cidx_v)
        pltpu.async_copy(ctab.at[cidx_v], crows_v, sem).wait()
        pltpu.sync_copy(crows_v, cout.at[pl.ds(cbase, CPW)])

    return gather_kernel(word_tab, widx, city_tab, cidx)


def _mm(a, b):
    return lax.dot_general(a, b, (((1,), (0,)), ((), ())),
                           preferred_element_type=jnp.float32)


def _ln_f(x, g, b):
    mu = jnp.mean(x, axis=0, keepdims=True)
    var = jnp.mean((x - mu) ** 2, axis=0, keepdims=True)
    return (x - mu) / jnp.sqrt(var + 1e-6) * g + b


def _tc_body(wrow_r, numT_r, crow_r, trkT_r, othT_r, pos2_r,
             linWT_r, linb_r, ttabT_r, ltabT_r, htabT_r, stabT_r, ctabT_r,
             wqv_r, l1g_r, l1b_r, w1T_r, b1_r, w2T_r, b2_r, l2g_r, l2b_r,
             dW1T_r, db1_r, dW2T_r, db2_r, out_r):
    f32 = jnp.float32
    nv2 = _mm(linWT_r[:], numT_r[:]) + linb_r[:]
    cT = jnp.transpose(crow_r[:])
    city2 = jnp.concatenate([cT[:, 0:TB], cT[:, TB:2 * TB]], axis=0)
    trkT = trkT_r[:]
    i20 = lax.broadcasted_iota(jnp.int32, (20, TB), 0)
    counts = (trkT[0:1, :] == i20).astype(f32)
    for j in range(1, 5):
        counts = counts + (trkT[j:j + 1, :] == i20).astype(f32)
    truck2 = _mm(ttabT_r[:], counts) * 0.2
    othT = othT_r[:]
    i50 = lax.broadcasted_iota(jnp.int32, (50, TB), 0)
    cat2 = _mm(ctabT_r[:], (othT[0:1, :] == i50).astype(f32))

    def pick3(lab, tabT_r):
        return jnp.where(lab == 0, tabT_r[:, 0:1],
                         jnp.where(lab == 1, tabT_r[:, 1:2], tabT_r[:, 2:3]))

    lcl2 = pick3(othT[1:2, :], ltabT_r)
    hand2 = pick3(othT[2:3, :], htabT_r)
    sec2 = pick3(othT[3:4, :], stabT_r)

    e2 = jnp.transpose(wrow_r[:]) + pos2_r[:]
    qv = _mm(wqv_r[:], e2)
    bf16 = jnp.bfloat16
    e2b = e2.astype(bf16)
    q2b = qv[0:D, :].astype(bf16)
    v2b = qv[D:2 * D, :].astype(bf16)
    ao_blocks = []
    for l in range(L):
        qlb = q2b[:, l * TB:(l + 1) * TB]
        srows = []
        for m in range(L):
            emb = e2b[:, m * TB:(m + 1) * TB]
            srows.append(
                jnp.sum(qlb * emb, axis=0, keepdims=True).astype(f32))
        s_l = jnp.concatenate(srows, axis=0)
        s_l = s_l - jnp.max(s_l, axis=0, keepdims=True)
        p_l = jnp.exp(s_l)
        a_l = (p_l / jnp.sum(p_l, axis=0, keepdims=True)).astype(bf16)
        ao_l = a_l[0:1, :] * v2b[:, 0:TB]
        for m in range(1, L):
            ao_l = ao_l + a_l[m:m + 1, :] * v2b[:, m * TB:(m + 1) * TB]
        ao_blocks.append(ao_l)
    ao2 = jnp.concatenate(ao_blocks, axis=1).astype(f32)
    h = _ln_f(e2 + ao2, l1g_r[:], l1b_r[:])
    ffo = _mm(w2T_r[:], jnp.maximum(_mm(w1T_r[:], h) + b1_r[:], 0.0)) + b2_r[:]
    enc = _ln_f(h + ffo, l2g_r[:], l2b_r[:])
    desc2 = jnp.concatenate(
        [enc[:, l * TB:(l + 1) * TB] for l in range(L)], axis=0)

    out2 = jnp.concatenate(
        [nv2, city2, truck2, lcl2, hand2, sec2, cat2, desc2], axis=0)
    ssum = out2[0:D, :]
    ssq = out2[0:D, :] * out2[0:D, :]
    for f in range(1, CONCAT // D):
        s = out2[D * f:D * (f + 1), :]
        ssum = ssum + s
        ssq = ssq + s * s
    fm = 0.5 * (ssum * ssum - ssq)
    hid = jnp.maximum(_mm(dW1T_r[:], out2) + db1_r[:], 0.0)
    dnn = _mm(dW2T_r[:], hid) + db2_r[:]
    z = 0.5 * (dnn + fm)
    nrm = jnp.sqrt(jnp.sum(z * z, axis=0, keepdims=True))
    out_r[:] = z / jnp.maximum(nrm, 1e-12)


def _tc_call(x2, numT, cityT, trkT, othT, consts, nb):
    bt = lambda shp: pl.BlockSpec(shp, lambda i: (0, 0))
    in_specs = [
        pl.BlockSpec((TW, D), lambda i: (i, 0)),
        pl.BlockSpec((NUMF, TB), lambda i: (0, i)),
        pl.BlockSpec((2 * TB, D), lambda i: (i, 0)),
        pl.BlockSpec((5, TB), lambda i: (0, i)),
        pl.BlockSpec((4, TB), lambda i: (0, i)),
    ] + [bt(c.shape) for c in consts]
    return pl.pallas_call(
        _tc_body,
        grid=(nb // TB,),
        in_specs=in_specs,
        out_specs=pl.BlockSpec((D, TB), lambda i: (0, i)),
        out_shape=jax.ShapeDtypeStruct((D, nb), jnp.float32),
    )(x2, numT, cityT, trkT, othT, *consts)


def kernel(cargo_numerical_features, cargo_city_labels, cargo_truck_type_labels,
           cargo_category_labels, cargo_is_lcl, cargo_handling_type,
           cargo_security_tran, cargo_describe,
           lin_W, lin_b, city_tab, truck_tab, lcl_tab, hand_tab, sec_tab,
           cat_tab, word_tab, pos_emb, Wq, Wk, Wv, Wo, ln1_g, ln1_b,
           ffn_W1, ffn_b1, ffn_W2, ffn_b2, ln2_g, ln2_b,
           dnn_W1, dnn_b1, dnn_W2, dnn_b2):
    pos2 = jnp.broadcast_to(pos_emb.T[:, :, None], (D, L, TB)).reshape(D, TW)
    col = lambda v: v.reshape(-1, 1)
    sc = 1.0 / np.sqrt(D)
    wqv = jnp.concatenate([(Wq @ Wk.T).T * sc, (Wv @ Wo).T], axis=0)
    consts = [
        pos2,
        (lin_W * (1.0 / np.sqrt(NUMF))).T, col(lin_b),
        truck_tab.T, lcl_tab.T, hand_tab.T, sec_tab.T, cat_tab.T,
        wqv, col(ln1_g), col(ln1_b),
        ffn_W1.T, col(ffn_b1), ffn_W2.T, col(ffn_b2), col(ln2_g), col(ln2_b),
        dnn_W1.T, col(dnn_b1), dnn_W2.T, col(dnn_b2),
    ]
    numT = cargo_numerical_features.T
    trkT = cargo_truck_type_labels.astype(jnp.int32).T
    othT = jnp.concatenate(
        [cargo_category_labels, cargo_is_lcl, cargo_handling_type,
         cargo_security_tran], axis=1).astype(jnp.int32).T
    NCK = 4
    BC = B // NCK
    NTC = BC // TB
    widx_all = cargo_describe.astype(jnp.int32)
    widx_all = widx_all.reshape(NCK * NTC, TB, L).transpose(0, 2, 1)
    widx_all = widx_all.reshape(NCK, BC * L)
    cidx_all = cargo_city_labels.astype(jnp.int32)
    cidx_all = cidx_all.reshape(NCK * NTC, TB, 2).transpose(0, 2, 1)
    cidx_all = cidx_all.reshape(NCK, 2 * BC)
    pieces = []
    for c in range(NCK):
        wrows, crows = _sc_gather(word_tab, widx_all[c], city_tab,
                                  cidx_all[c], BC)
        pieces.append(_tc_call(
            wrows, numT[:, c * BC:(c + 1) * BC], crows,
            trkT[:, c * BC:(c + 1) * BC], othT[:, c * BC:(c + 1) * BC],
            consts, BC))
    return jnp.concatenate(pieces, axis=1).T


# speedup vs baseline: 1.0389x; 1.0389x over previous
import functools

import jax
import jax.numpy as jnp
import numpy as np
from jax import lax
from jax.experimental import pallas as pl
from jax.experimental.pallas import tpu as pltpu
from jax.experimental.pallas import tpu_sc as plsc

B = 16384
L = 20
D = 32
NUMF = 26
HID = 256
CONCAT = 896
TB = 128
NT = B // TB
TW = TB * L


def _sc_gather(word_tab, widx, city_tab, cidx, nb):
    info = plsc.get_sparse_core_info()
    NC, NS = info.num_cores, info.num_subcores
    NW = NC * NS
    WPW = (nb * L) // NW
    CH = min(WPW, 2048)
    NCH = WPW // CH
    CPW = (2 * nb) // NW

    mesh = plsc.VectorSubcoreMesh(core_axis_name="c", subcore_axis_name="s")

    @functools.partial(
        pl.kernel,
        mesh=mesh,
        compiler_params=pltpu.CompilerParams(use_tc_tiling_on_sc=False),
        out_type=[
            jax.ShapeDtypeStruct((nb * L, D), jnp.float32),
            jax.ShapeDtypeStruct((2 * nb, D), jnp.float32),
        ],
        scratch_types=[
            pltpu.VMEM((CH,), jnp.int32),
            pltpu.VMEM((CH, D), jnp.float32),
            pltpu.VMEM((CPW,), jnp.int32),
            pltpu.VMEM((CPW, D), jnp.float32),
            pltpu.SemaphoreType.DMA,
        ],
    )
    def gather_kernel(wtab, wi, ctab, ci, wout, cout,
                      idx_v, rows_v, cidx_v, crows_v, sem):
        wid = lax.axis_index("s") * NC + lax.axis_index("c")
        base = wid * WPW
        for c in range(NCH):
            off = base + c * CH
            pltpu.sync_copy(wi.at[pl.ds(off, CH)], idx_v)
            pltpu.async_copy(wtab.at[idx_v], rows_v, sem).wait()
            pltpu.sync_copy(rows_v, wout.at[pl.ds(off, CH)])
        cbase = wid * CPW
        pltpu.sync_copy(ci.at[pl.ds(cbase, CPW)], cidx_v)
        pltpu.async_copy(ctab.at[cidx_v], crows_v, sem).wait()
        pltpu.sync_copy(crows_v, cout.at[pl.ds(cbase, CPW)])

    return gather_kernel(word_tab, widx, city_tab, cidx)


def _mm(a, b):
    return lax.dot_general(a, b, (((1,), (0,)), ((), ())),
                           preferred_element_type=jnp.float32)


def _ln_f(x, g, b):
    mu = jnp.mean(x, axis=0, keepdims=True)
    var = jnp.mean((x - mu) ** 2, axis=0, keepdims=True)
    return (x - mu) / jnp.sqrt(var + 1e-6) * g + b


def _one_group(wr, numT, cr, trkT, othT, pos2,
               linWT, linb, ttabT, ltabT, htabT, stabT, ctabT,
               wqv, l1g, l1b, w1T, b1, w2T, b2, l2g, l2b,
               dW1T, db1, dW2T, db2):
    f32 = jnp.float32
    nv2 = _mm(linWT, numT) + linb
    cT = jnp.transpose(cr)
    city2 = jnp.concatenate([cT[:, 0:TB], cT[:, TB:2 * TB]], axis=0)
    i20 = lax.broadcasted_iota(jnp.int32, (20, TB), 0)
    counts = (trkT[0:1, :] == i20).astype(f32)
    for j in range(1, 5):
        counts = counts + (trkT[j:j + 1, :] == i20).astype(f32)
    truck2 = _mm(ttabT, counts) * 0.2
    i50 = lax.broadcasted_iota(jnp.int32, (50, TB), 0)
    cat2 = _mm(ctabT, (othT[0:1, :] == i50).astype(f32))

    def pick3(lab, tabT):
        return jnp.where(lab == 0, tabT[:, 0:1],
                         jnp.where(lab == 1, tabT[:, 1:2], tabT[:, 2:3]))

    lcl2 = pick3(othT[1:2, :], ltabT)
    hand2 = pick3(othT[2:3, :], htabT)
    sec2 = pick3(othT[3:4, :], stabT)

    e2 = jnp.transpose(wr) + pos2
    qv = _mm(wqv, e2)
    q2 = qv[0:D, :]
    v2 = qv[D:2 * D, :]
    ao_blocks = []
    for l in range(L):
        ql = q2[:, l * TB:(l + 1) * TB]
        srows = []
        for m in range(L):
            em = e2[:, m * TB:(m + 1) * TB]
            srows.append(jnp.sum(ql * em, axis=0, keepdims=True))
        s_l = jnp.concatenate(srows, axis=0)
        s_l = s_l - jnp.max(s_l, axis=0, keepdims=True)
        p_l = jnp.exp(s_l)
        a_l = p_l / jnp.sum(p_l, axis=0, keepdims=True)
        ao_l = a_l[0:1, :] * v2[:, 0:TB]
        for m in range(1, L):
            ao_l = ao_l + a_l[m:m + 1, :] * v2[:, m * TB:(m + 1) * TB]
        ao_blocks.append(ao_l)
    ao2 = jnp.concatenate(ao_blocks, axis=1)
    h = _ln_f(e2 + ao2, l1g, l1b)
    ffo = _mm(w2T, jnp.maximum(_mm(w1T, h) + b1, 0.0)) + b2
    enc = _ln_f(h + ffo, l2g, l2b)
    desc2 = jnp.concatenate(
        [enc[:, l * TB:(l + 1) * TB] for l in range(L)], axis=0)

    out2 = jnp.concatenate(
        [nv2, city2, truck2, lcl2, hand2, sec2, cat2, desc2], axis=0)
    ssum = out2[0:D, :]
    ssq = out2[0:D, :] * out2[0:D, :]
    for f in range(1, CONCAT // D):
        s = out2[D * f:D * (f + 1), :]
        ssum = ssum + s
        ssq = ssq + s * s
    fm = 0.5 * (ssum * ssum - ssq)
    hid = jnp.maximum(_mm(dW1T, out2) + db1, 0.0)
    dnn = _mm(dW2T, hid) + db2
    z = 0.5 * (dnn + fm)
    nrm = jnp.sqrt(jnp.sum(z * z, axis=0, keepdims=True))
    return z / jnp.maximum(nrm, 1e-12)


G = 2
GB = G * TB


def _tc_body(wrow_r, numT_r, crow_r, trkT_r, othT_r, pos2_r,
             linWT_r, linb_r, ttabT_r, ltabT_r, htabT_r, stabT_r, ctabT_r,
             wqv_r, l1g_r, l1b_r, w1T_r, b1_r, w2T_r, b2_r, l2g_r, l2b_r,
             dW1T_r, db1_r, dW2T_r, db2_r, out_r):
    consts = (pos2_r[:], linWT_r[:], linb_r[:], ttabT_r[:], ltabT_r[:],
              htabT_r[:], stabT_r[:], ctabT_r[:], wqv_r[:], l1g_r[:],
              l1b_r[:], w1T_r[:], b1_r[:], w2T_r[:], b2_r[:], l2g_r[:],
              l2b_r[:], dW1T_r[:], db1_r[:], dW2T_r[:], db2_r[:])
    for g in range(G):
        out_r[:, g * TB:(g + 1) * TB] = _one_group(
            wrow_r[g * TW:(g + 1) * TW, :],
            numT_r[:, g * TB:(g + 1) * TB],
            crow_r[g * 2 * TB:(g + 1) * 2 * TB, :],
            trkT_r[:, g * TB:(g + 1) * TB],
            othT_r[:, g * TB:(g + 1) * TB],
            *consts)


def _tc_call(x2, numT, cityT, trkT, othT, consts, nb):
    bt = lambda shp: pl.BlockSpec(shp, lambda i: (0, 0))
    in_specs = [
        pl.BlockSpec((G * TW, D), lambda i: (i, 0)),
        pl.BlockSpec((NUMF, GB), lambda i: (0, i)),
        pl.BlockSpec((G * 2 * TB, D), lambda i: (i, 0)),
        pl.BlockSpec((5, GB), lambda i: (0, i)),
        pl.BlockSpec((4, GB), lambda i: (0, i)),
    ] + [bt(c.shape) for c in consts]
    return pl.pallas_call(
        _tc_body,
        grid=(nb // GB,),
        in_specs=in_specs,
        out_specs=pl.BlockSpec((D, GB), lambda i: (0, i)),
        out_shape=jax.ShapeDtypeStruct((D, nb), jnp.float32),
    )(x2, numT, cityT, trkT, othT, *consts)


def kernel(cargo_numerical_features, cargo_city_labels, cargo_truck_type_labels,
           cargo_category_labels, cargo_is_lcl, cargo_handling_type,
           cargo_security_tran, cargo_describe,
           lin_W, lin_b, city_tab, truck_tab, lcl_tab, hand_tab, sec_tab,
           cat_tab, word_tab, pos_emb, Wq, Wk, Wv, Wo, ln1_g, ln1_b,
           ffn_W1, ffn_b1, ffn_W2, ffn_b2, ln2_g, ln2_b,
           dnn_W1, dnn_b1, dnn_W2, dnn_b2):
    pos2 = jnp.broadcast_to(pos_emb.T[:, :, None], (D, L, TB)).reshape(D, TW)
    col = lambda v: v.reshape(-1, 1)
    sc = 1.0 / np.sqrt(D)
    wqv = jnp.concatenate([(Wq @ Wk.T).T * sc, (Wv @ Wo).T], axis=0)
    consts = [
        pos2,
        (lin_W * (1.0 / np.sqrt(NUMF))).T, col(lin_b),
        truck_tab.T, lcl_tab.T, hand_tab.T, sec_tab.T, cat_tab.T,
        wqv, col(ln1_g), col(ln1_b),
        ffn_W1.T, col(ffn_b1), ffn_W2.T, col(ffn_b2), col(ln2_g), col(ln2_b),
        dnn_W1.T, col(dnn_b1), dnn_W2.T, col(dnn_b2),
    ]
    numT = cargo_numerical_features.T
    trkT = cargo_truck_type_labels.astype(jnp.int32).T
    othT = jnp.concatenate(
        [cargo_category_labels, cargo_is_lcl, cargo_handling_type,
         cargo_security_tran], axis=1).astype(jnp.int32).T
    NCK = 4
    BC = B // NCK
    NTC = BC // TB
    widx_all = cargo_describe.astype(jnp.int32)
    widx_all = widx_all.reshape(NCK * NTC, TB, L).transpose(0, 2, 1)
    widx_all = widx_all.reshape(NCK, BC * L)
    cidx_all = cargo_city_labels.astype(jnp.int32)
    cidx_all = cidx_all.reshape(NCK * NTC, TB, 2).transpose(0, 2, 1)
    cidx_all = cidx_all.reshape(NCK, 2 * BC)
    pieces = []
    for c in range(NCK):
        wrows, crows = _sc_gather(word_tab, widx_all[c], city_tab,
                                  cidx_all[c], BC)
        pieces.append(_tc_call(
            wrows, numT[:, c * BC:(c + 1) * BC], crows,
            trkT[:, c * BC:(c + 1) * BC], othT[:, c * BC:(c + 1) * BC],
            consts, BC))
    return jnp.concatenate(pieces, axis=1).T
